# Initial kernel scaffold; baseline (speedup 1.0000x reference)
import jax
import jax.numpy as jnp
from jax.experimental import pallas as pl


def kernel(x, edge_index, W1, Wmu, Wlv):
    raise NotImplementedError("write your pallas kernel here")


# speedup vs baseline: 23.6609x; 23.6609x over previous
import functools

import jax
import jax.numpy as jnp
from jax import lax
from jax.experimental import pallas as pl
from jax.experimental.pallas import tpu as pltpu
from jax.experimental.pallas import tpu_sc as plsc

_NC, _NS = 2, 16
_NW = _NC * _NS
_B = 100


def _sc_mesh():
    return plsc.VectorSubcoreMesh(core_axis_name="c", subcore_axis_name="s",
                                  num_cores=_NC, num_subcores=_NS)


@functools.cache
def _deg_kernel(n, e):
    epw = e // _NW

    @functools.partial(
        pl.kernel,
        out_type=jax.ShapeDtypeStruct((_NW, n), jnp.float32),
        mesh=_sc_mesh(),
        scratch_types=[pltpu.VMEM((epw,), jnp.int32),
                       pltpu.VMEM((n,), jnp.float32)],
        compiler_params=pltpu.CompilerParams(needs_layout_passes=False),
    )
    def deg(dst_hbm, out_hbm, dst_v, hist_v):
        c = lax.axis_index("c")
        s = lax.axis_index("s")
        wid = s * _NC + c
        pltpu.sync_copy(dst_hbm.at[pl.ds(wid * epw, epw)], dst_v)
        zeros16 = jnp.zeros((16,), jnp.float32)

        def zbody(i, carry):
            hist_v[pl.ds(i * 16, 16)] = zeros16
            return carry

        lax.fori_loop(0, n // 16, zbody, 0)
        ones16 = jnp.ones((16,), jnp.float32)

        def abody(i, carry):
            plsc.addupdate_scatter(hist_v, [dst_v[pl.ds(i * 16, 16)]], ones16)
            return carry

        lax.fori_loop(0, epw // 16, abody, 0)
        pltpu.sync_copy(hist_v, out_hbm.at[wid])

    return deg


@functools.cache
def _scatter_kernel(n, e, w):
    epw = e // _NW
    nbw = epw // _B
    rpt = n // _NS

    @functools.partial(
        pl.kernel,
        out_type=jax.ShapeDtypeStruct((_NC, n, w), jnp.float32),
        mesh=_sc_mesh(),
        scratch_types=[
            pltpu.VMEM((nbw, _B), jnp.int32),
            pltpu.VMEM((nbw, _B), jnp.int32),
            pltpu.VMEM((_B, w), jnp.float32),
            pltpu.SemaphoreType.DMA,
            pltpu.VMEM_SHARED((n, w), jnp.float32),
        ],
        compiler_params=pltpu.CompilerParams(needs_layout_passes=False,
                                             use_tc_tiling_on_sc=False),
    )
    def scat(src2_hbm, dst2_hbm, y_hbm, zero_hbm, out_hbm,
             src_v, dst_v, rows_v, sem, acc):
        c = lax.axis_index("c")
        s = lax.axis_index("s")
        wid = s * _NC + c
        pltpu.sync_copy(zero_hbm, acc.at[pl.ds(s * rpt, rpt)])
        pltpu.sync_copy(src2_hbm.at[pl.ds(wid * nbw, nbw)], src_v)
        pltpu.sync_copy(dst2_hbm.at[pl.ds(wid * nbw, nbw)], dst_v)
        plsc.subcore_barrier()

        def body(j, carry):
            pltpu.async_copy(y_hbm.at[src_v.at[j]], rows_v, sem).wait()
            pltpu.sync_copy(rows_v, acc.at[dst_v.at[j]], add=True)
            return carry

        lax.fori_loop(0, nbw, body, 0)
        plsc.subcore_barrier()
        pltpu.sync_copy(acc.at[pl.ds(s * rpt, rpt)],
                        out_hbm.at[c, pl.ds(s * rpt, rpt)])

    return scat


def _tc1_body(deg_ref, x_ref, w1_ref, dinv_ref, y1_ref):
    total = 1.0 + jnp.sum(deg_ref[...], axis=0)
    dinv = lax.rsqrt(total)[:, None]
    dinv_ref[...] = dinv
    y1_ref[...] = jnp.dot(x_ref[...], w1_ref[...],
                          preferred_element_type=jnp.float32) * dinv


def _tc2_body(sp_ref, y1_ref, dinv_ref, wc_ref, y2_ref):
    dinv = dinv_ref[...]
    h = jnp.maximum((sp_ref[0] + sp_ref[1] + y1_ref[...]) * dinv, 0.0)
    y2_ref[...] = jnp.dot(h, wc_ref[...],
                          preferred_element_type=jnp.float32) * dinv


def _tc3_body(sp_ref, y2_ref, dinv_ref, mlv_ref):
    mlv_ref[...] = (sp_ref[0] + sp_ref[1] + y2_ref[...]) * dinv_ref[...]


def _dec_body(mu_r_ref, mu_c_ref, out_ref):
    p = lax.dot_general(mu_r_ref[...], mu_c_ref[...],
                        (((1,), (1,)), ((), ())),
                        preferred_element_type=jnp.float32)
    out_ref[...] = jax.nn.sigmoid(p)


def kernel(x, edge_index, W1, Wmu, Wlv):
    n, _ = x.shape
    hdim = W1.shape[1]
    ldim = Wmu.shape[1]
    e = edge_index.shape[1]
    src2 = edge_index[0].reshape(e // _B, _B)
    dst2 = edge_index[1].reshape(e // _B, _B)

    deg_parts = _deg_kernel(n, e)(edge_index[1])

    dinv, y1 = pl.pallas_call(
        _tc1_body,
        out_shape=[jax.ShapeDtypeStruct((n, 1), jnp.float32),
                   jax.ShapeDtypeStruct((n, hdim), jnp.float32)],
    )(deg_parts, x, W1)

    s1 = _scatter_kernel(n, e, hdim)(
        src2, dst2, y1, jnp.zeros((n // _NS, hdim), jnp.float32))

    wc = jnp.concatenate([Wmu, Wlv], axis=1)
    y2 = pl.pallas_call(
        _tc2_body,
        out_shape=jax.ShapeDtypeStruct((n, 2 * ldim), jnp.float32),
    )(s1, y1, dinv, wc)

    s2 = _scatter_kernel(n, e, 2 * ldim)(
        src2, dst2, y2, jnp.zeros((n // _NS, 2 * ldim), jnp.float32))

    mlv = pl.pallas_call(
        _tc3_body,
        out_shape=jax.ShapeDtypeStruct((n, 2 * ldim), jnp.float32),
    )(s2, y2, dinv)

    mu = mlv[:, :ldim]
    logvar = mlv[:, ldim:]

    br, bc = 1000, 2048
    adj = pl.pallas_call(
        _dec_body,
        grid=(n // br, pl.cdiv(n, bc)),
        in_specs=[pl.BlockSpec((br, ldim), lambda i, j: (i, 0)),
                  pl.BlockSpec((bc, ldim), lambda i, j: (j, 0))],
        out_specs=pl.BlockSpec((br, bc), lambda i, j: (i, j)),
        out_shape=jax.ShapeDtypeStruct((n, n), jnp.float32),
    )(mu, mu)

    return adj, mu, logvar


# speedup vs baseline: 27.6984x; 1.1706x over previous
import functools

import jax
import jax.numpy as jnp
from jax import lax
from jax.experimental import pallas as pl
from jax.experimental.pallas import tpu as pltpu
from jax.experimental.pallas import tpu_sc as plsc

_NC, _NS = 2, 16
_NW = _NC * _NS
_B = 125


def _sc_mesh():
    return plsc.VectorSubcoreMesh(core_axis_name="c", subcore_axis_name="s",
                                  num_cores=_NC, num_subcores=_NS)


@functools.cache
def _deg_kernel(n, e):
    epw = e // _NW

    @functools.partial(
        pl.kernel,
        out_type=jax.ShapeDtypeStruct((_NW, n), jnp.float32),
        mesh=_sc_mesh(),
        scratch_types=[pltpu.VMEM((epw,), jnp.int32),
                       pltpu.VMEM((n,), jnp.float32)],
        compiler_params=pltpu.CompilerParams(needs_layout_passes=False),
    )
    def deg(dst_hbm, out_hbm, dst_v, hist_v):
        c = lax.axis_index("c")
        s = lax.axis_index("s")
        wid = s * _NC + c
        pltpu.sync_copy(dst_hbm.at[pl.ds(wid * epw, epw)], dst_v)
        zeros16 = jnp.zeros((16,), jnp.float32)

        def zbody(i, carry):
            hist_v[pl.ds(i * 16, 16)] = zeros16
            return carry

        lax.fori_loop(0, n // 16, zbody, 0)
        ones16 = jnp.ones((16,), jnp.float32)

        def abody(i, carry):
            plsc.addupdate_scatter(hist_v, [dst_v[pl.ds(i * 16, 16)]], ones16)
            return carry

        lax.fori_loop(0, epw // 16, abody, 0)
        pltpu.sync_copy(hist_v, out_hbm.at[wid])

    return deg


@functools.cache
def _scatter_kernel(n, e, w):
    epw = e // _NW
    nbw = epw // _B
    rpt = n // _NS

    @functools.partial(
        pl.kernel,
        out_type=jax.ShapeDtypeStruct((_NC, n, w), jnp.float32),
        mesh=_sc_mesh(),
        scratch_types=[
            pltpu.VMEM((nbw, _B), jnp.int32),
            pltpu.VMEM((nbw, _B), jnp.int32),
            pltpu.VMEM((_B, w), jnp.float32),
            pltpu.VMEM((_B, w), jnp.float32),
            pltpu.SemaphoreType.DMA,
            pltpu.SemaphoreType.DMA,
            pltpu.VMEM_SHARED((n, w), jnp.float32),
        ],
        compiler_params=pltpu.CompilerParams(needs_layout_passes=False,
                                             use_tc_tiling_on_sc=False),
    )
    def scat(src2_hbm, dst2_hbm, y_hbm, zero_hbm, out_hbm,
             src_v, dst_v, rows_a, rows_b, sem_a, sem_b, acc):
        c = lax.axis_index("c")
        s = lax.axis_index("s")
        wid = s * _NC + c
        pltpu.sync_copy(zero_hbm, acc.at[pl.ds(s * rpt, rpt)])
        pltpu.sync_copy(src2_hbm.at[pl.ds(wid * nbw, nbw)], src_v)
        pltpu.sync_copy(dst2_hbm.at[pl.ds(wid * nbw, nbw)], dst_v)
        plsc.subcore_barrier()

        npairs = nbw // 2
        pltpu.async_copy(y_hbm.at[src_v.at[0]], rows_a, sem_a)

        def body(p, carry):
            j0 = 2 * p
            pltpu.make_async_copy(y_hbm.at[src_v.at[j0]], rows_a, sem_a).wait()
            pltpu.async_copy(y_hbm.at[src_v.at[j0 + 1]], rows_b, sem_b)
            pltpu.sync_copy(rows_a, acc.at[dst_v.at[j0]], add=True)
            pltpu.make_async_copy(y_hbm.at[src_v.at[j0 + 1]], rows_b,
                                  sem_b).wait()

            @pl.when(p + 1 < npairs)
            def _():
                pltpu.async_copy(y_hbm.at[src_v.at[j0 + 2]], rows_a, sem_a)

            pltpu.sync_copy(rows_b, acc.at[dst_v.at[j0 + 1]], add=True)
            return carry

        lax.fori_loop(0, npairs, body, 0)
        plsc.subcore_barrier()
        pltpu.sync_copy(acc.at[pl.ds(s * rpt, rpt)],
                        out_hbm.at[c, pl.ds(s * rpt, rpt)])

    return scat


def _tc1_body(deg_ref, x_ref, w1_ref, dinv_ref, y1_ref):
    total = 1.0 + jnp.sum(deg_ref[...], axis=0)
    dinv = lax.rsqrt(total)[:, None]
    dinv_ref[...] = dinv
    y1_ref[...] = jnp.dot(x_ref[...], w1_ref[...],
                          preferred_element_type=jnp.float32) * dinv


def _tc2_body(sp_ref, y1_ref, dinv_ref, wc_ref, y2_ref):
    dinv = dinv_ref[...]
    h = jnp.maximum((sp_ref[0] + sp_ref[1] + y1_ref[...]) * dinv, 0.0)
    y2_ref[...] = jnp.dot(h, wc_ref[...],
                          preferred_element_type=jnp.float32) * dinv


def _tc3_body(sp_ref, y2_ref, dinv_ref, mlv_ref):
    mlv_ref[...] = (sp_ref[0] + sp_ref[1] + y2_ref[...]) * dinv_ref[...]


def _dec_body(mu_r_ref, mu_c_ref, out_ref):
    p = lax.dot_general(mu_r_ref[...], mu_c_ref[...],
                        (((1,), (1,)), ((), ())),
                        preferred_element_type=jnp.float32)
    out_ref[...] = jax.nn.sigmoid(p)


def kernel(x, edge_index, W1, Wmu, Wlv):
    n, _ = x.shape
    hdim = W1.shape[1]
    ldim = Wmu.shape[1]
    e = edge_index.shape[1]
    src2 = edge_index[0].reshape(e // _B, _B)
    dst2 = edge_index[1].reshape(e // _B, _B)

    deg_parts = _deg_kernel(n, e)(edge_index[1])

    dinv, y1 = pl.pallas_call(
        _tc1_body,
        out_shape=[jax.ShapeDtypeStruct((n, 1), jnp.float32),
                   jax.ShapeDtypeStruct((n, hdim), jnp.float32)],
    )(deg_parts, x, W1)

    s1 = _scatter_kernel(n, e, hdim)(
        src2, dst2, y1, jnp.zeros((n // _NS, hdim), jnp.float32))

    wc = jnp.concatenate([Wmu, Wlv], axis=1)
    y2 = pl.pallas_call(
        _tc2_body,
        out_shape=jax.ShapeDtypeStruct((n, 2 * ldim), jnp.float32),
    )(s1, y1, dinv, wc)

    s2 = _scatter_kernel(n, e, 2 * ldim)(
        src2, dst2, y2, jnp.zeros((n // _NS, 2 * ldim), jnp.float32))

    mlv = pl.pallas_call(
        _tc3_body,
        out_shape=jax.ShapeDtypeStruct((n, 2 * ldim), jnp.float32),
    )(s2, y2, dinv)

    mu = mlv[:, :ldim]
    logvar = mlv[:, ldim:]

    br, bc = 1000, 2048
    adj = pl.pallas_call(
        _dec_body,
        grid=(n // br, pl.cdiv(n, bc)),
        in_specs=[pl.BlockSpec((br, ldim), lambda i, j: (i, 0)),
                  pl.BlockSpec((bc, ldim), lambda i, j: (j, 0))],
        out_specs=pl.BlockSpec((br, bc), lambda i, j: (i, j)),
        out_shape=jax.ShapeDtypeStruct((n, n), jnp.float32),
    )(mu, mu)

    return adj, mu, logvar


# speedup vs baseline: 32.4056x; 1.1699x over previous
import functools

import jax
import jax.numpy as jnp
from jax import lax
from jax.experimental import pallas as pl
from jax.experimental.pallas import tpu as pltpu
from jax.experimental.pallas import tpu_sc as plsc

_NC, _NS = 2, 16
_NW = _NC * _NS
_B = 500


def _sc_mesh():
    return plsc.VectorSubcoreMesh(core_axis_name="c", subcore_axis_name="s",
                                  num_cores=_NC, num_subcores=_NS)


@functools.cache
def _deg_kernel(n, e):
    epw = e // _NW

    @functools.partial(
        pl.kernel,
        out_type=jax.ShapeDtypeStruct((_NW, n), jnp.float32),
        mesh=_sc_mesh(),
        scratch_types=[pltpu.VMEM((epw,), jnp.int32),
                       pltpu.VMEM((n,), jnp.float32)],
        compiler_params=pltpu.CompilerParams(needs_layout_passes=False),
    )
    def deg(dst_hbm, out_hbm, dst_v, hist_v):
        c = lax.axis_index("c")
        s = lax.axis_index("s")
        wid = s * _NC + c
        pltpu.sync_copy(dst_hbm.at[pl.ds(wid * epw, epw)], dst_v)
        zeros16 = jnp.zeros((16,), jnp.float32)

        def zbody(i, carry):
            hist_v[pl.ds(i * 16, 16)] = zeros16
            return carry

        lax.fori_loop(0, n // 16, zbody, 0)
        ones16 = jnp.ones((16,), jnp.float32)

        def abody(i, carry):
            plsc.addupdate_scatter(hist_v, [dst_v[pl.ds(i * 16, 16)]], ones16)
            return carry

        lax.fori_loop(0, epw // 16, abody, 0)
        pltpu.sync_copy(hist_v, out_hbm.at[wid])

    return deg


@functools.cache
def _scatter_kernel(n, e, w):
    epw = e // _NW
    nbw = epw // _B
    rpt = n // _NS

    @functools.partial(
        pl.kernel,
        out_type=jax.ShapeDtypeStruct((_NC, n, w), jnp.float32),
        mesh=_sc_mesh(),
        scratch_types=[
            pltpu.VMEM((nbw, _B), jnp.int32),
            pltpu.VMEM((nbw, _B), jnp.int32),
            pltpu.VMEM((_B, w), jnp.float32),
            pltpu.VMEM((_B, w), jnp.float32),
            pltpu.SemaphoreType.DMA,
            pltpu.SemaphoreType.DMA,
            pltpu.VMEM_SHARED((n, w), jnp.float32),
        ],
        compiler_params=pltpu.CompilerParams(needs_layout_passes=False,
                                             use_tc_tiling_on_sc=False),
    )
    def scat(src2_hbm, dst2_hbm, y_hbm, zero_hbm, out_hbm,
             src_v, dst_v, rows_a, rows_b, sem_a, sem_b, acc):
        c = lax.axis_index("c")
        s = lax.axis_index("s")
        wid = s * _NC + c
        pltpu.sync_copy(zero_hbm, acc.at[pl.ds(s * rpt, rpt)])
        pltpu.sync_copy(src2_hbm.at[pl.ds(wid * nbw, nbw)], src_v)
        pltpu.sync_copy(dst2_hbm.at[pl.ds(wid * nbw, nbw)], dst_v)
        plsc.subcore_barrier()

        npairs = nbw // 2
        pltpu.async_copy(y_hbm.at[src_v.at[0]], rows_a, sem_a)

        def body(p, carry):
            j0 = 2 * p
            pltpu.make_async_copy(y_hbm.at[src_v.at[j0]], rows_a, sem_a).wait()
            pltpu.async_copy(y_hbm.at[src_v.at[j0 + 1]], rows_b, sem_b)
            pltpu.sync_copy(rows_a, acc.at[dst_v.at[j0]], add=True)
            pltpu.make_async_copy(y_hbm.at[src_v.at[j0 + 1]], rows_b,
                                  sem_b).wait()

            @pl.when(p + 1 < npairs)
            def _():
                pltpu.async_copy(y_hbm.at[src_v.at[j0 + 2]], rows_a, sem_a)

            pltpu.sync_copy(rows_b, acc.at[dst_v.at[j0 + 1]], add=True)
            return carry

        lax.fori_loop(0, npairs, body, 0)
        plsc.subcore_barrier()
        pltpu.sync_copy(acc.at[pl.ds(s * rpt, rpt)],
                        out_hbm.at[c, pl.ds(s * rpt, rpt)])

    return scat


def _tc1_body(deg_ref, x_ref, w1_ref, dinv_ref, y1_ref):
    total = 1.0 + jnp.sum(deg_ref[...], axis=0)
    dinv = lax.rsqrt(total)[:, None]
    dinv_ref[...] = dinv
    y1_ref[...] = jnp.dot(x_ref[...], w1_ref[...],
                          preferred_element_type=jnp.float32) * dinv


def _tc2_body(sp_ref, y1_ref, dinv_ref, wc_ref, y2_ref):
    dinv = dinv_ref[...]
    h = jnp.maximum((sp_ref[0] + sp_ref[1] + y1_ref[...]) * dinv, 0.0)
    y2_ref[...] = jnp.dot(h, wc_ref[...],
                          preferred_element_type=jnp.float32) * dinv


def _tc3_body(sp_ref, y2_ref, dinv_ref, mlv_ref):
    mlv_ref[...] = (sp_ref[0] + sp_ref[1] + y2_ref[...]) * dinv_ref[...]


def _dec_body(mu_r_ref, mu_c_ref, out_ref):
    p = lax.dot_general(mu_r_ref[...], mu_c_ref[...],
                        (((1,), (1,)), ((), ())),
                        preferred_element_type=jnp.float32)
    out_ref[...] = jax.nn.sigmoid(p)


def kernel(x, edge_index, W1, Wmu, Wlv):
    n, _ = x.shape
    hdim = W1.shape[1]
    ldim = Wmu.shape[1]
    e = edge_index.shape[1]
    src2 = edge_index[0].reshape(e // _B, _B)
    dst2 = edge_index[1].reshape(e // _B, _B)

    deg_parts = _deg_kernel(n, e)(edge_index[1])

    dinv, y1 = pl.pallas_call(
        _tc1_body,
        out_shape=[jax.ShapeDtypeStruct((n, 1), jnp.float32),
                   jax.ShapeDtypeStruct((n, hdim), jnp.float32)],
    )(deg_parts, x, W1)

    s1 = _scatter_kernel(n, e, hdim)(
        src2, dst2, y1, jnp.zeros((n // _NS, hdim), jnp.float32))

    wc = jnp.concatenate([Wmu, Wlv], axis=1)
    y2 = pl.pallas_call(
        _tc2_body,
        out_shape=jax.ShapeDtypeStruct((n, 2 * ldim), jnp.float32),
    )(s1, y1, dinv, wc)

    s2 = _scatter_kernel(n, e, 2 * ldim)(
        src2, dst2, y2, jnp.zeros((n // _NS, 2 * ldim), jnp.float32))

    mlv = pl.pallas_call(
        _tc3_body,
        out_shape=jax.ShapeDtypeStruct((n, 2 * ldim), jnp.float32),
    )(s2, y2, dinv)

    mu = mlv[:, :ldim]
    logvar = mlv[:, ldim:]

    br, bc = 1000, 2048
    adj = pl.pallas_call(
        _dec_body,
        grid=(n // br, pl.cdiv(n, bc)),
        in_specs=[pl.BlockSpec((br, ldim), lambda i, j: (i, 0)),
                  pl.BlockSpec((bc, ldim), lambda i, j: (j, 0))],
        out_specs=pl.BlockSpec((br, bc), lambda i, j: (i, j)),
        out_shape=jax.ShapeDtypeStruct((n, n), jnp.float32),
    )(mu, mu)

    return adj, mu, logvar


# speedup vs baseline: 33.6426x; 1.0382x over previous
import functools

import jax
import jax.numpy as jnp
from jax import lax
from jax.experimental import pallas as pl
from jax.experimental.pallas import tpu as pltpu
from jax.experimental.pallas import tpu_sc as plsc

_NC, _NS = 2, 16
_NW = _NC * _NS
_B = 500


def _sc_mesh():
    return plsc.VectorSubcoreMesh(core_axis_name="c", subcore_axis_name="s",
                                  num_cores=_NC, num_subcores=_NS)


@functools.cache
def _deg_kernel(n, e):
    epw = e // _NW

    @functools.partial(
        pl.kernel,
        out_type=jax.ShapeDtypeStruct((_NW, n), jnp.float32),
        mesh=_sc_mesh(),
        scratch_types=[pltpu.VMEM((epw,), jnp.int32),
                       pltpu.VMEM((n,), jnp.float32)],
        compiler_params=pltpu.CompilerParams(needs_layout_passes=False),
    )
    def deg(dst_hbm, out_hbm, dst_v, hist_v):
        c = lax.axis_index("c")
        s = lax.axis_index("s")
        wid = s * _NC + c
        pltpu.sync_copy(dst_hbm.at[pl.ds(wid * epw, epw)], dst_v)
        zeros16 = jnp.zeros((16,), jnp.float32)

        def zbody(i, carry):
            hist_v[pl.ds(i * 16, 16)] = zeros16
            return carry

        lax.fori_loop(0, n // 16, zbody, 0)
        ones16 = jnp.ones((16,), jnp.float32)

        def abody(i, carry):
            plsc.addupdate_scatter(hist_v, [dst_v[pl.ds(i * 16, 16)]], ones16)
            return carry

        lax.fori_loop(0, epw // 16, abody, 0)
        pltpu.sync_copy(hist_v, out_hbm.at[wid])

    return deg


@functools.cache
def _scatter_kernel(n, e, w):
    epw = e // _NW
    nbw = epw // _B
    rpt = n // _NS

    @functools.partial(
        pl.kernel,
        out_type=jax.ShapeDtypeStruct((_NC, n, w), jnp.float32),
        mesh=_sc_mesh(),
        scratch_types=[
            pltpu.VMEM((nbw, _B), jnp.int32),
            pltpu.VMEM((nbw, _B), jnp.int32),
            pltpu.VMEM((_B, w), jnp.float32),
            pltpu.VMEM((_B, w), jnp.float32),
            pltpu.SemaphoreType.DMA,
            pltpu.SemaphoreType.DMA,
            pltpu.VMEM_SHARED((n, w), jnp.float32),
        ],
        compiler_params=pltpu.CompilerParams(needs_layout_passes=False,
                                             use_tc_tiling_on_sc=False),
    )
    def scat(src2_hbm, dst2_hbm, y_hbm, zero_hbm, out_hbm,
             src_v, dst_v, rows_a, rows_b, sem_a, sem_b, acc):
        c = lax.axis_index("c")
        s = lax.axis_index("s")
        wid = s * _NC + c
        pltpu.sync_copy(zero_hbm, acc.at[pl.ds(s * rpt, rpt)])
        pltpu.sync_copy(src2_hbm.at[pl.ds(wid * nbw, nbw)], src_v)
        pltpu.sync_copy(dst2_hbm.at[pl.ds(wid * nbw, nbw)], dst_v)
        plsc.subcore_barrier()

        npairs = nbw // 2
        pltpu.async_copy(y_hbm.at[src_v.at[0]], rows_a, sem_a)

        def body(p, carry):
            j0 = 2 * p
            pltpu.make_async_copy(y_hbm.at[src_v.at[j0]], rows_a, sem_a).wait()
            pltpu.async_copy(y_hbm.at[src_v.at[j0 + 1]], rows_b, sem_b)
            pltpu.sync_copy(rows_a, acc.at[dst_v.at[j0]], add=True)
            pltpu.make_async_copy(y_hbm.at[src_v.at[j0 + 1]], rows_b,
                                  sem_b).wait()

            @pl.when(p + 1 < npairs)
            def _():
                pltpu.async_copy(y_hbm.at[src_v.at[j0 + 2]], rows_a, sem_a)

            pltpu.sync_copy(rows_b, acc.at[dst_v.at[j0 + 1]], add=True)
            return carry

        lax.fori_loop(0, npairs, body, 0)
        plsc.subcore_barrier()
        pltpu.sync_copy(acc.at[pl.ds(s * rpt, rpt)],
                        out_hbm.at[c, pl.ds(s * rpt, rpt)])

    return scat


def _tc1_body(deg_ref, x_ref, w1_ref, dinv_ref, y1_ref):
    total = 1.0 + jnp.sum(deg_ref[...], axis=0)
    dinv = lax.rsqrt(total)[:, None]
    dinv_ref[...] = dinv
    y1_ref[...] = jnp.dot(x_ref[...], w1_ref[...],
                          preferred_element_type=jnp.float32) * dinv


def _tc2_body(sp_ref, y1_ref, dinv_ref, wc_ref, y2_ref):
    dinv = dinv_ref[...]
    h = jnp.maximum((sp_ref[0] + sp_ref[1] + y1_ref[...]) * dinv, 0.0)
    y2_ref[...] = jnp.dot(h, wc_ref[...],
                          preferred_element_type=jnp.float32) * dinv


def _tc3_body(sp_ref, y2_ref, dinv_ref, mlv_ref):
    mlv_ref[...] = (sp_ref[0] + sp_ref[1] + y2_ref[...]) * dinv_ref[...]


def _dec_body(mu_r_ref, mu_c_ref, out_ref):
    p = lax.dot_general(mu_r_ref[...], mu_c_ref[...],
                        (((1,), (1,)), ((), ())),
                        preferred_element_type=jnp.float32)
    out_ref[...] = jax.nn.sigmoid(p)


def kernel(x, edge_index, W1, Wmu, Wlv):
    n, _ = x.shape
    hdim = W1.shape[1]
    ldim = Wmu.shape[1]
    e = edge_index.shape[1]
    src2 = edge_index[0].reshape(e // _B, _B)
    dst2 = edge_index[1].reshape(e // _B, _B)

    deg_parts = _deg_kernel(n, e)(edge_index[1])

    dinv, y1 = pl.pallas_call(
        _tc1_body,
        out_shape=[jax.ShapeDtypeStruct((n, 1), jnp.float32),
                   jax.ShapeDtypeStruct((n, hdim), jnp.float32)],
    )(deg_parts, x, W1)

    s1 = _scatter_kernel(n, e, hdim)(
        src2, dst2, y1, jnp.zeros((n // _NS, hdim), jnp.float32))

    wc = jnp.concatenate([Wmu, Wlv], axis=1)
    y2 = pl.pallas_call(
        _tc2_body,
        out_shape=jax.ShapeDtypeStruct((n, 2 * ldim), jnp.float32),
    )(s1, y1, dinv, wc)

    s2 = _scatter_kernel(n, e, 2 * ldim)(
        src2, dst2, y2, jnp.zeros((n // _NS, 2 * ldim), jnp.float32))

    mlv = pl.pallas_call(
        _tc3_body,
        out_shape=jax.ShapeDtypeStruct((n, 2 * ldim), jnp.float32),
    )(s2, y2, dinv)

    mu = mlv[:, :ldim]
    logvar = mlv[:, ldim:]

    br = 400
    adj = pl.pallas_call(
        _dec_body,
        grid=(n // br,),
        in_specs=[pl.BlockSpec((br, ldim), lambda i: (i, 0)),
                  pl.BlockSpec((n, ldim), lambda i: (0, 0))],
        out_specs=pl.BlockSpec((br, n), lambda i: (i, 0)),
        out_shape=jax.ShapeDtypeStruct((n, n), jnp.float32),
    )(mu, mu)

    return adj, mu, logvar
